# baseline (speedup 1.0000x reference)
import functools

import jax
import jax.numpy as jnp
from jax import lax
from jax.experimental import pallas as pl
from jax.experimental.pallas import tpu as pltpu
from jax.experimental.pallas import tpu_sc as plsc

NUM_CLASS = 100000
DATASET_NUM = 8
CTX_DIM = 512
B = 1024
SEQ = 77
REP = 32
NSTATIC = 72

NC = 2
NS = 16
NW = NC * NS
BPW = B // NW
HALF = BPW // 2

SC_PLANES = NSTATIC // NC
STAGE_PER_TILE = SC_PLANES // 12

_mesh = plsc.VectorSubcoreMesh(core_axis_name="c", subcore_axis_name="s")


@functools.partial(
    pl.kernel,
    out_type=jax.ShapeDtypeStruct((SEQ, B, CTX_DIM), jnp.float32),
    mesh=_mesh,
    scratch_types=[
        pltpu.VMEM((1, 1, 2 * BPW), jnp.int32),
        pltpu.VMEM((HALF, 4, CTX_DIM), jnp.float32),
        pltpu.VMEM((HALF, 4, CTX_DIM), jnp.float32),
        pltpu.VMEM((HALF, 1, CTX_DIM), jnp.float32),
        pltpu.VMEM((HALF, CTX_DIM), jnp.float32),
        pltpu.VMEM((HALF, CTX_DIM), jnp.float32),
        pltpu.VMEM_SHARED((SC_PLANES, REP, CTX_DIM), jnp.float32),
        pltpu.SemaphoreType.DMA,
        pltpu.SemaphoreType.DMA,
        pltpu.SemaphoreType.DMA,
        pltpu.SemaphoreType.DMA,
        pltpu.SemaphoreType.DMA,
        pltpu.SemaphoreType.DMA,
    ],
)
def _assemble(idx_h, cls_h, dm_h, tmpl_h, out_h,
              idx_v, rows_a, rows_b, drows_v, stage0, stage1, tmpl_s,
              gsem_a, gsem_b, dsem, ssem0, ssem1, tsem):
    cid = lax.axis_index("c")
    sid = lax.axis_index("s")
    wid = cid * NS + sid
    base = wid * BPW

    pltpu.sync_copy(idx_h.at[wid], idx_v.at[0])
    iv0 = idx_v[0, 0, pl.ds(0, HALF)]
    iv1 = idx_v[0, 0, pl.ds(HALF, HALF)]
    dv0 = idx_v[0, 0, pl.ds(2 * HALF, HALF)]
    dv1 = idx_v[0, 0, pl.ds(3 * HALF, HALF)]
    g0 = pltpu.async_copy(cls_h.at[iv0], rows_a, gsem_a)
    g1 = pltpu.async_copy(cls_h.at[iv1], rows_b, gsem_b)
    gd0 = pltpu.async_copy(dm_h.at[dv0], drows_v, dsem)

    static_cps = []
    for s in range(SC_PLANES):
        p = cid * SC_PLANES + s
        t = p + jnp.where(p < 5, 0, jnp.where(p < 7, 4, 5))
        for h in range(2):
            static_cps.append(pltpu.async_copy(
                tmpl_h.at[cid * SC_PLANES + s],
                out_h.at[t, pl.ds(sid * 2 * BPW + h * BPW, BPW)],
                tsem))

    def repack(stage, src, j):
        def row(r, _):
            for ch in range(CTX_DIM // 16):
                stage[r, pl.ds(ch * 16, 16)] = src[r, j, pl.ds(ch * 16, 16)]
            return 0
        lax.fori_loop(0, HALF, row, 0)

    def drain_stage(stage, sem):
        pltpu.make_async_copy(
            stage, out_h.at[0, pl.ds(base, HALF)], sem).wait()

    n = 0

    def do_plane(t, src, j, c):
        nonlocal n
        stage, sem = (stage0, ssem0) if n % 2 == 0 else (stage1, ssem1)
        if n >= 2:
            drain_stage(stage, sem)
        repack(stage, src, j)
        pltpu.async_copy(stage, out_h.at[t, pl.ds(base + c * HALF, HALF)], sem)
        n += 1

    g0.wait()
    for j in range(4):
        do_plane(5 + j, rows_a, j, 0)
    gd0.wait()
    do_plane(11, drows_v, 0, 0)
    gd1 = pltpu.async_copy(dm_h.at[dv1], drows_v, dsem)
    g1.wait()
    for j in range(4):
        do_plane(5 + j, rows_b, j, 1)
    gd1.wait()
    do_plane(11, drows_v, 0, 1)
    drain_stage(stage0, ssem0)
    drain_stage(stage1, ssem1)

    for cp in static_cps:
        cp.wait()


def kernel(label, domain, clsctx, dmctx, token_prefix_domain,
           token_intermediate_domain, token_suffix_domain):
    idx = jnp.concatenate(
        [label.astype(jnp.int32).reshape(NW, 1, BPW),
         domain.astype(jnp.int32).reshape(NW, 1, BPW)], axis=2)
    static_rows = jnp.concatenate(
        [token_prefix_domain, token_intermediate_domain,
         token_suffix_domain], axis=1)
    tmpl = jnp.broadcast_to(static_rows.reshape(NSTATIC, 1, CTX_DIM),
                            (NSTATIC, REP, CTX_DIM))
    res = _assemble(idx, clsctx, dmctx, tmpl)
    return jnp.transpose(res, (1, 0, 2))


# speedup vs baseline: 40.4462x; 40.4462x over previous
import functools

import jax
import jax.numpy as jnp
from jax import lax
from jax.experimental import pallas as pl
from jax.experimental.pallas import tpu as pltpu
from jax.experimental.pallas import tpu_sc as plsc

NUM_CLASS = 100000
DATASET_NUM = 8
CTX_DIM = 512
B = 1024
SEQ = 77
REP = 32
NSTATIC = 72

NC = 2
NS = 16
NW = NC * NS
BPW = B // NW
HALF = BPW // 2

SC_PLANES = NSTATIC // NC
STAGE_PER_TILE = SC_PLANES // 12

_mesh = plsc.VectorSubcoreMesh(core_axis_name="c", subcore_axis_name="s")


@functools.partial(
    pl.kernel,
    out_type=jax.ShapeDtypeStruct((SEQ, B, CTX_DIM), jnp.float32),
    mesh=_mesh,
    scratch_types=[
        pltpu.VMEM((1, 1, BPW), jnp.int32),
        pltpu.VMEM((1, 1, BPW), jnp.int32),
        pltpu.VMEM((HALF, 4, CTX_DIM), jnp.float32),
        pltpu.VMEM((HALF, 4, CTX_DIM), jnp.float32),
        pltpu.VMEM((HALF, 1, CTX_DIM), jnp.float32),
        pltpu.VMEM((HALF, CTX_DIM), jnp.float32),
        pltpu.VMEM((HALF, CTX_DIM), jnp.float32),
        pltpu.VMEM_SHARED((SC_PLANES, REP, CTX_DIM), jnp.float32),
        pltpu.SemaphoreType.DMA,
        pltpu.SemaphoreType.DMA,
        pltpu.SemaphoreType.DMA,
        pltpu.SemaphoreType.DMA,
        pltpu.SemaphoreType.DMA,
        pltpu.SemaphoreType.DMA,
    ],
)
def _assemble(lab_h, dom_h, cls_h, dm_h, tmpl_h, out_h,
              idx_v, didx_v, rows_a, rows_b, drows_v, stage0, stage1, tmpl_s,
              gsem_a, gsem_b, dsem, ssem0, ssem1, tsem):
    cid = lax.axis_index("c")
    sid = lax.axis_index("s")
    wid = cid * NS + sid
    base = wid * BPW

    pltpu.sync_copy(lab_h.at[wid], idx_v.at[0])
    pltpu.sync_copy(dom_h.at[wid], didx_v.at[0])
    iv0 = idx_v[0, 0, pl.ds(0, HALF)]
    iv1 = idx_v[0, 0, pl.ds(HALF, HALF)]
    dv0 = didx_v[0, 0, pl.ds(0, HALF)]
    dv1 = didx_v[0, 0, pl.ds(HALF, HALF)]
    g0 = pltpu.async_copy(cls_h.at[iv0], rows_a, gsem_a)
    g1 = pltpu.async_copy(cls_h.at[iv1], rows_b, gsem_b)
    gd0 = pltpu.async_copy(dm_h.at[dv0], drows_v, dsem)

    @pl.when(sid < 12)
    def _stage_tmpl():
        src = tmpl_h.at[pl.ds(cid * SC_PLANES + sid * STAGE_PER_TILE,
                              STAGE_PER_TILE)]
        pltpu.sync_copy(src, tmpl_s.at[pl.ds(sid * STAGE_PER_TILE,
                                             STAGE_PER_TILE)])
    plsc.subcore_barrier()

    def fire_static(s, _):
        p = cid * SC_PLANES + s
        t = p + jnp.where(p < 5, 0, jnp.where(p < 7, 4, 5))
        for h in range(2):
            pltpu.async_copy(
                tmpl_s.at[s],
                out_h.at[t, pl.ds(sid * 2 * BPW + h * BPW, BPW)],
                tsem)
        return 0
    lax.fori_loop(0, SC_PLANES, fire_static, 0)

    def repack(stage, src, j):
        def row(r, _):
            for ch in range(CTX_DIM // 16):
                stage[r, pl.ds(ch * 16, 16)] = src[r, j, pl.ds(ch * 16, 16)]
            return 0
        lax.fori_loop(0, HALF, row, 0)

    def drain_stage(stage, sem):
        pltpu.make_async_copy(
            stage, out_h.at[0, pl.ds(base, HALF)], sem).wait()

    n = 0

    def do_plane(t, src, j, c):
        nonlocal n
        stage, sem = (stage0, ssem0) if n % 2 == 0 else (stage1, ssem1)
        if n >= 2:
            drain_stage(stage, sem)
        repack(stage, src, j)
        pltpu.async_copy(stage, out_h.at[t, pl.ds(base + c * HALF, HALF)], sem)
        n += 1

    g0.wait()
    for j in range(4):
        do_plane(5 + j, rows_a, j, 0)
    gd0.wait()
    do_plane(11, drows_v, 0, 0)
    gd1 = pltpu.async_copy(dm_h.at[dv1], drows_v, dsem)
    g1.wait()
    for j in range(4):
        do_plane(5 + j, rows_b, j, 1)
    gd1.wait()
    do_plane(11, drows_v, 0, 1)
    drain_stage(stage0, ssem0)
    drain_stage(stage1, ssem1)

    def drain_static(s, _):
        for h in range(2):
            pltpu.make_async_copy(
                tmpl_s.at[0], out_h.at[0, pl.ds(0, BPW)], tsem).wait()
        return 0
    lax.fori_loop(0, SC_PLANES, drain_static, 0)


def kernel(label, domain, clsctx, dmctx, token_prefix_domain,
           token_intermediate_domain, token_suffix_domain):
    lab = label.astype(jnp.int32).reshape(NW, 1, BPW)
    dom = domain.astype(jnp.int32).reshape(NW, 1, BPW)
    static_rows = jnp.concatenate(
        [token_prefix_domain, token_intermediate_domain,
         token_suffix_domain], axis=1)
    tmpl = jnp.broadcast_to(static_rows.reshape(NSTATIC, 1, CTX_DIM),
                            (NSTATIC, REP, CTX_DIM))
    res = _assemble(lab, dom, clsctx, dmctx, tmpl)
    return jnp.transpose(res, (1, 0, 2))


# speedup vs baseline: 42.8186x; 1.0587x over previous
import functools

import jax
import jax.numpy as jnp
from jax import lax
from jax.experimental import pallas as pl
from jax.experimental.pallas import tpu as pltpu
from jax.experimental.pallas import tpu_sc as plsc

NUM_CLASS = 100000
DATASET_NUM = 8
CTX_DIM = 512
B = 1024
SEQ = 77

NC = 2
NS = 16
NW = NC * NS
BPW = B // NW
HALF = BPW // 2

_mesh = plsc.VectorSubcoreMesh(core_axis_name="c", subcore_axis_name="s")


@functools.partial(
    pl.kernel,
    out_type=[jax.ShapeDtypeStruct((4, B, CTX_DIM), jnp.float32),
              jax.ShapeDtypeStruct((1, B, CTX_DIM), jnp.float32)],
    mesh=_mesh,
    scratch_types=[
        pltpu.VMEM((1, 1, BPW), jnp.int32),
        pltpu.VMEM((1, 1, BPW), jnp.int32),
        pltpu.VMEM((HALF, 4, CTX_DIM), jnp.float32),
        pltpu.VMEM((HALF, 4, CTX_DIM), jnp.float32),
        pltpu.VMEM((HALF, 1, CTX_DIM), jnp.float32),
        pltpu.VMEM((HALF, CTX_DIM), jnp.float32),
        pltpu.VMEM((HALF, CTX_DIM), jnp.float32),
        pltpu.SemaphoreType.DMA,
        pltpu.SemaphoreType.DMA,
        pltpu.SemaphoreType.DMA,
        pltpu.SemaphoreType.DMA,
        pltpu.SemaphoreType.DMA,
    ],
)
def _gather_planes(lab_h, dom_h, cls_h, dm_h, cls_o, dom_o,
                   idx_v, didx_v, rows_a, rows_b, drows_v, stage0, stage1,
                   gsem_a, gsem_b, dsem, ssem0, ssem1):
    cid = lax.axis_index("c")
    sid = lax.axis_index("s")
    wid = cid * NS + sid
    base = wid * BPW

    pltpu.sync_copy(lab_h.at[wid], idx_v.at[0])
    pltpu.sync_copy(dom_h.at[wid], didx_v.at[0])
    iv0 = idx_v[0, 0, pl.ds(0, HALF)]
    iv1 = idx_v[0, 0, pl.ds(HALF, HALF)]
    dv0 = didx_v[0, 0, pl.ds(0, HALF)]
    dv1 = didx_v[0, 0, pl.ds(HALF, HALF)]
    g0 = pltpu.async_copy(cls_h.at[iv0], rows_a, gsem_a)
    g1 = pltpu.async_copy(cls_h.at[iv1], rows_b, gsem_b)
    gd0 = pltpu.async_copy(dm_h.at[dv0], drows_v, dsem)

    def repack(stage, src, j):
        def row(r, _):
            for ch in range(CTX_DIM // 16):
                stage[r, pl.ds(ch * 16, 16)] = src[r, j, pl.ds(ch * 16, 16)]
            return 0
        lax.fori_loop(0, HALF, row, 0)

    def drain_stage(stage, sem):
        pltpu.make_async_copy(
            stage, cls_o.at[0, pl.ds(base, HALF)], sem).wait()

    n = 0

    def do_plane(out_ref, t, src, j, c):
        nonlocal n
        stage, sem = (stage0, ssem0) if n % 2 == 0 else (stage1, ssem1)
        if n >= 2:
            drain_stage(stage, sem)
        repack(stage, src, j)
        pltpu.async_copy(
            stage, out_ref.at[t, pl.ds(base + c * HALF, HALF)], sem)
        n += 1

    g0.wait()
    for j in range(4):
        do_plane(cls_o, j, rows_a, j, 0)
    gd0.wait()
    do_plane(dom_o, 0, drows_v, 0, 0)
    gd1 = pltpu.async_copy(dm_h.at[dv1], drows_v, dsem)
    g1.wait()
    for j in range(4):
        do_plane(cls_o, j, rows_b, j, 1)
    gd1.wait()
    do_plane(dom_o, 0, drows_v, 0, 1)
    drain_stage(stage0, ssem0)
    drain_stage(stage1, ssem1)


def kernel(label, domain, clsctx, dmctx, token_prefix_domain,
           token_intermediate_domain, token_suffix_domain):
    lab = label.astype(jnp.int32).reshape(NW, 1, BPW)
    dom = domain.astype(jnp.int32).reshape(NW, 1, BPW)
    cls_pl, dom_pl = _gather_planes(lab, dom, clsctx, dmctx)
    cls_ctx = jnp.transpose(cls_pl, (1, 0, 2))
    dom_ctx = jnp.transpose(dom_pl, (1, 0, 2))
    prefix = jnp.broadcast_to(token_prefix_domain, (B, 5, CTX_DIM))
    inter = jnp.broadcast_to(token_intermediate_domain, (B, 2, CTX_DIM))
    suffix = jnp.broadcast_to(token_suffix_domain, (B, 65, CTX_DIM))
    return jnp.concatenate([prefix, cls_ctx, inter, dom_ctx, suffix], axis=1)


# speedup vs baseline: 47.1586x; 1.1014x over previous
import functools

import jax
import jax.numpy as jnp
from jax import lax
from jax.experimental import pallas as pl
from jax.experimental.pallas import tpu as pltpu
from jax.experimental.pallas import tpu_sc as plsc

NUM_CLASS = 100000
DATASET_NUM = 8
CTX_DIM = 512
B = 1024
SEQ = 77

NC = 2
NS = 16
NW = NC * NS
BPW = B // NW
HALF = BPW // 2

_mesh = plsc.VectorSubcoreMesh(core_axis_name="c", subcore_axis_name="s")


@functools.partial(
    pl.kernel,
    out_type=[jax.ShapeDtypeStruct((4, B, CTX_DIM), jnp.float32),
              jax.ShapeDtypeStruct((1, B, CTX_DIM), jnp.float32)],
    mesh=_mesh,
    scratch_types=[
        pltpu.VMEM((1, 1, BPW), jnp.int32),
        pltpu.VMEM((1, 1, BPW), jnp.int32),
        pltpu.VMEM((HALF, 4, CTX_DIM), jnp.float32),
        pltpu.VMEM((HALF, 4, CTX_DIM), jnp.float32),
        pltpu.VMEM((HALF, 1, CTX_DIM), jnp.float32),
        pltpu.VMEM((HALF, CTX_DIM), jnp.float32),
        pltpu.VMEM((HALF, CTX_DIM), jnp.float32),
        pltpu.SemaphoreType.DMA,
        pltpu.SemaphoreType.DMA,
        pltpu.SemaphoreType.DMA,
        pltpu.SemaphoreType.DMA,
        pltpu.SemaphoreType.DMA,
    ],
)
def _gather_planes(lab_h, dom_h, cls_h, dm_h, cls_o, dom_o,
                   idx_v, didx_v, rows_a, rows_b, drows_v, stage0, stage1,
                   gsem_a, gsem_b, dsem, ssem0, ssem1):
    cid = lax.axis_index("c")
    sid = lax.axis_index("s")
    wid = cid * NS + sid
    base = wid * BPW

    pltpu.sync_copy(lab_h.at[wid], idx_v.at[0])
    pltpu.sync_copy(dom_h.at[wid], didx_v.at[0])
    iv0 = idx_v[0, 0, pl.ds(0, HALF)]
    iv1 = idx_v[0, 0, pl.ds(HALF, HALF)]
    dv0 = didx_v[0, 0, pl.ds(0, HALF)]
    dv1 = didx_v[0, 0, pl.ds(HALF, HALF)]
    g0 = pltpu.async_copy(cls_h.at[iv0], rows_a, gsem_a)
    g1 = pltpu.async_copy(cls_h.at[iv1], rows_b, gsem_b)
    gd0 = pltpu.async_copy(dm_h.at[dv0], drows_v, dsem)

    def repack(stage, src, j):
        def row(r, _):
            for ch in range(CTX_DIM // 16):
                stage[r, pl.ds(ch * 16, 16)] = src[r, j, pl.ds(ch * 16, 16)]
            return 0
        lax.fori_loop(0, HALF, row, 0)

    def drain_stage(stage, sem):
        pltpu.make_async_copy(
            stage, cls_o.at[0, pl.ds(base, HALF)], sem).wait()

    n = 0

    def do_plane(out_ref, t, src, j, c):
        nonlocal n
        stage, sem = (stage0, ssem0) if n % 2 == 0 else (stage1, ssem1)
        if n >= 2:
            drain_stage(stage, sem)
        repack(stage, src, j)
        pltpu.async_copy(
            stage, out_ref.at[t, pl.ds(base + c * HALF, HALF)], sem)
        n += 1

    g0.wait()
    for j in range(4):
        do_plane(cls_o, j, rows_a, j, 0)
    gd0.wait()
    do_plane(dom_o, 0, drows_v, 0, 0)
    gd1 = pltpu.async_copy(dm_h.at[dv1], drows_v, dsem)
    g1.wait()
    for j in range(4):
        do_plane(cls_o, j, rows_b, j, 1)
    gd1.wait()
    do_plane(dom_o, 0, drows_v, 0, 1)
    drain_stage(stage0, ssem0)
    drain_stage(stage1, ssem1)


def kernel(label, domain, clsctx, dmctx, token_prefix_domain,
           token_intermediate_domain, token_suffix_domain):
    lab = label.astype(jnp.int32).reshape(NW, 1, BPW)
    dom = domain.astype(jnp.int32).reshape(NW, 1, BPW)
    cls_pl, dom_pl = _gather_planes(lab, dom, clsctx, dmctx)
    cls_ctx = jnp.transpose(cls_pl, (1, 0, 2))
    dom_ctx = jnp.transpose(dom_pl, (1, 0, 2))
    prefix = jnp.broadcast_to(token_prefix_domain, (B, 5, CTX_DIM))
    inter = jnp.broadcast_to(token_intermediate_domain, (B, 2, CTX_DIM))
    suffix = jnp.broadcast_to(token_suffix_domain, (B, 65, CTX_DIM))
    base = jnp.concatenate(
        [prefix, jnp.zeros((B, 4, CTX_DIM), jnp.float32), inter,
         jnp.zeros((B, 1, CTX_DIM), jnp.float32), suffix], axis=1)
    out = lax.dynamic_update_slice(base, cls_ctx, (0, 5, 0))
    return lax.dynamic_update_slice(out, dom_ctx, (0, 11, 0))
